# baseline (speedup 1.0000x reference)
import functools

import jax
import jax.numpy as jnp
import numpy as np
from jax import lax
from jax.experimental import pallas as pl
from jax.experimental.pallas import tpu as pltpu
from jax.experimental.pallas import tpu_sc as plsc

_GRID = (256, 256, 32)
_NUM_CLASSES = 21
_MIN_B = np.array([-25.6, -25.6, -2.0], np.float32)
_MAX_B = np.array([25.6, 25.6, 4.4], np.float32)

_NC = 2
_NS = 16
_NW = _NC * _NS
_LANES = 16


def _sc_scatter_add(ref, idxs, x, mask):
  plsc.addupdate_scatter(ref, idxs, x, mask=mask)


def _sc_store_scatter(ref, idxs, x, mask):
  plsc.store_scatter(ref, idxs, x, mask=mask)


def _sc_load_gather(ref, idx):
  return plsc.load_gather(ref, [idx])


def _sc_scan_count(x, mask):
  return plsc.scan_count(x, mask=mask)


def _sc_cumsum(x):
  return plsc.cumsum(x)


def _axis_index(name):
  return lax.axis_index(name)


def _sync_copy(src, dst):
  pltpu.sync_copy(src, dst)


def _make_body(X, Y, Z, C, n_pad, chunk, clcap, slcap):
  assert X % _NW == 0 and Y % 2 == 0 and n_pad % chunk == 0
  assert chunk % _LANES == 0
  xpw = X // _NW
  yh = Y // 2
  zc = Z * C
  n_chunks = n_pad // chunk
  vecs_per_chunk = chunk // _LANES

  minb = [float(v) for v in _MIN_B]
  maxb = [float(v) for v in _MAX_B]
  vs = (np.asarray(_MAX_B) - np.asarray(_MIN_B)) / np.asarray(
      (X, Y, Z), np.float32)
  inv_vs = [float(np.float32(1.0) / v) for v in vs]

  lane_iota = lambda: lax.iota(jnp.int32, _LANES)

  def body(map_hbm, pts_hbm, w_hbm, out_hbm,
           acc, clist, slist, pbuf, filt):
    wid = _axis_index("s") * _NC + _axis_index("c")
    x_lo = wid * xpw
    win_lo = x_lo - 1

    _sync_copy(w_hbm, filt)
    v0 = filt[pl.ds(0, _LANES)]
    v0 = 1.0 / (1.0 + jnp.exp(-v0))
    v0 = jnp.where(lane_iota() == 13, 1.0, v0)
    filt[pl.ds(0, _LANES)] = v0
    v1 = filt[pl.ds(_LANES, _LANES)]
    v1 = 1.0 / (1.0 + jnp.exp(-v1))
    filt[pl.ds(_LANES, _LANES)] = v1

    def chunk_body(ci, n):
      _sync_copy(pts_hbm.at[ci], pbuf)

      def vec_body(i, n):
        off = i * _LANES
        xv = pbuf[0, pl.ds(off, _LANES)]
        yv = pbuf[1, pl.ds(off, _LANES)]
        zv = pbuf[2, pl.ds(off, _LANES)]
        cv = pbuf[3, pl.ds(off, _LANES)]
        fx = (xv - minb[0]) * inv_vs[0]
        fy = (yv - minb[1]) * inv_vs[1]
        fz = (zv - minb[2]) * inv_vs[2]
        ix = jnp.clip(fx.astype(jnp.int32), 0, X - 1)
        iy = jnp.clip(fy.astype(jnp.int32), 0, Y - 1)
        iz = jnp.clip(fz.astype(jnp.int32), 0, Z - 1)
        ic = jnp.clip(cv.astype(jnp.int32), 0, C - 1)
        valid = ((xv >= minb[0]) & (xv < maxb[0])
                 & (yv >= minb[1]) & (yv < maxb[1])
                 & (zv >= minb[2]) & (zv < maxb[2]))
        m = valid & (ix >= win_lo) & (ix <= x_lo + xpw)
        code = ((ix - win_lo) << 18) | (iy << 10) | (iz << 5) | ic
        mi = m.astype(jnp.int32)
        pos = jnp.clip(n + _sc_cumsum(mi) - 1, 0, clcap - 1)
        _sc_store_scatter(clist, [pos], code, m)
        return n + jnp.sum(mi)

      return lax.fori_loop(0, vecs_per_chunk, vec_body, n)

    n_pts = lax.fori_loop(0, n_chunks, chunk_body, jnp.int32(0))

    def slab_body(s, _):
      def filt_body(i, ns):
        off = i * _LANES
        codes = clist[pl.ds(off, _LANES)]
        lm = (lane_iota() + off) < n_pts
        ixl = codes >> 18
        m = lm & (ixl >= s) & (ixl <= s + 2)
        mi = m.astype(jnp.int32)
        pos = jnp.clip(ns + _sc_cumsum(mi) - 1, 0, slcap - 1)
        _sc_store_scatter(slist, [pos], codes, m)
        return ns + jnp.sum(mi)

      n_vecs = (n_pts + _LANES - 1) // _LANES
      ns_pts = lax.fori_loop(0, n_vecs, filt_body, jnp.int32(0))
      sx = x_lo + s

      def half_body(h, _):
        _sync_copy(map_hbm.at[sx, pl.ds(h * yh, yh)], acc)
        ylo = h * yh

        def pt_body(i, _):
          off = i * _LANES
          codes = slist[pl.ds(off, _LANES)]
          lm = (lane_iota() + off) < ns_pts
          ixl = codes >> 18
          iy = (codes >> 10) & 0xFF
          iz = (codes >> 5) & 0x1F
          ic = codes & 0x1F
          ly = iy - ylo
          m0 = lm & (ly >= -1) & (ly <= yh)
          bidx = ly * zc + iz * C + ic
          cnts, _lastm = _sc_scan_count(bidx, m0)
          minc = jnp.min(jnp.where(m0, cnts, jnp.int32(2**30)))
          maxc = jnp.max(jnp.where(m0, cnts, jnp.int32(-2**30)))
          k9 = ((ixl - 1 - s) + 1) * 9
          icc = jnp.clip(ic, 0, C - 1)
          my = {dy: (ly + dy >= 0) & (ly + dy < yh) for dy in (-1, 0, 1)}
          mz = {dz: (iz + dz >= 0) & (iz + dz < Z) for dz in (-1, 0, 1)}
          taps = []
          for dy in (-1, 0, 1):
            lyt = jnp.clip(ly + dy, 0, yh - 1)
            for dz in (-1, 0, 1):
              izt = jnp.clip(iz + dz, 0, Z - 1)
              mt = m0 & my[dy] & mz[dz]
              wv = _sc_load_gather(
                  filt, jnp.clip(k9 + (1 - dy) * 3 + (1 - dz), 0, 31))
              taps.append((lyt, izt, wv, mt))

          def round_body(r, _):
            mr = cnts == r
            for lyt, izt, wv, mt in taps:
              _sc_scatter_add(acc, [lyt, izt, icc], wv, mt & mr)
            return 0

          lax.fori_loop(minc, maxc + 1, round_body, 0)
          return 0

        ns_vecs = (ns_pts + _LANES - 1) // _LANES
        lax.fori_loop(0, ns_vecs, pt_body, 0)
        _sync_copy(acc, out_hbm.at[sx, pl.ds(ylo, yh)])
        return 0

      lax.fori_loop(0, 2, half_body, 0)
      return 0

    lax.fori_loop(0, xpw, slab_body, 0)

  return body


def _make_kernel(X, Y, Z, C, n_pad, chunk, clcap, slcap):
  body = _make_body(X, Y, Z, C, n_pad, chunk, clcap, slcap)
  yh = Y // 2
  mesh = plsc.VectorSubcoreMesh(
      core_axis_name="c", subcore_axis_name="s", num_cores=_NC,
      num_subcores=_NS)
  return pl.kernel(
      body,
      out_type=jax.ShapeDtypeStruct((X, Y, Z, C), jnp.float32),
      mesh=mesh,
      scratch_types=[
          pltpu.VMEM((yh, Z, C), jnp.float32),
          pltpu.VMEM((clcap,), jnp.int32),
          pltpu.VMEM((slcap,), jnp.int32),
          pltpu.VMEM((4, chunk), jnp.float32),
          pltpu.VMEM((32,), jnp.float32),
      ],
      compiler_params=pltpu.CompilerParams(
          needs_layout_passes=False, use_tc_tiling_on_sc=False),
  )


@jax.jit
def kernel(current_map, point_cloud, weights):
  X, Y, Z, C = current_map.shape
  n = point_cloud.shape[0]
  chunk = 2048
  n_pad = ((n + chunk - 1) // chunk) * chunk
  pts = jnp.concatenate(
      [point_cloud,
       jnp.full((n_pad - n, 4), 1e30, point_cloud.dtype)], axis=0)
  pts4 = jnp.transpose(pts.reshape(n_pad // chunk, chunk, 4), (0, 2, 1))
  w_flat = jnp.concatenate(
      [weights.reshape(-1), jnp.zeros((32 - 27,), weights.dtype)])
  k = _make_kernel(X, Y, Z, C, n_pad, chunk, clcap=16000, slcap=8192)
  return k(current_map, pts4, w_flat)


# speedup vs baseline: 1.0686x; 1.0686x over previous
import functools

import jax
import jax.numpy as jnp
import numpy as np
from jax import lax
from jax.experimental import pallas as pl
from jax.experimental.pallas import tpu as pltpu
from jax.experimental.pallas import tpu_sc as plsc

_GRID = (256, 256, 32)
_NUM_CLASSES = 21
_MIN_B = np.array([-25.6, -25.6, -2.0], np.float32)
_MAX_B = np.array([25.6, 25.6, 4.4], np.float32)

_NC = 2
_NS = 16
_NW = _NC * _NS
_LANES = 16


def _sc_scatter_add(ref, idxs, x, mask):
  plsc.addupdate_scatter(ref, idxs, x, mask=mask)


def _sc_store_scatter(ref, idxs, x, mask):
  plsc.store_scatter(ref, idxs, x, mask=mask)


def _sc_load_gather(ref, idx):
  return plsc.load_gather(ref, [idx])


def _sc_scan_count(x, mask):
  return plsc.scan_count(x, mask=mask)


def _sc_cumsum(x):
  return plsc.cumsum(x)


def _axis_index(name):
  return lax.axis_index(name)


def _sync_copy(src, dst):
  pltpu.sync_copy(src, dst)


def _make_body(X, Y, Z, C, n_pad, chunk, clcap, slcap):
  assert X % _NW == 0 and Y % 2 == 0 and n_pad % chunk == 0
  assert chunk % _LANES == 0
  xpw = X // _NW
  yh = Y // 2
  zc = Z * C
  n_chunks = n_pad // chunk
  vecs_per_chunk = chunk // _LANES

  minb = [float(v) for v in _MIN_B]
  maxb = [float(v) for v in _MAX_B]
  vs = (np.asarray(_MAX_B) - np.asarray(_MIN_B)) / np.asarray(
      (X, Y, Z), np.float32)
  inv_vs = [float(np.float32(1.0) / v) for v in vs]

  lane_iota = lambda: lax.iota(jnp.int32, _LANES)

  reg = yh * zc

  def body(map_hbm, pts_hbm, w_hbm, out_hbm,
           acc, clist, slist, pbuf, filt):
    wid = _axis_index("s") * _NC + _axis_index("c")
    x_lo = wid * xpw
    win_lo = x_lo - 1

    _sync_copy(w_hbm, filt)
    v0 = filt[pl.ds(0, _LANES)]
    v0 = 1.0 / (1.0 + jnp.exp(-v0))
    v0 = jnp.where(lane_iota() == 13, 1.0, v0)
    filt[pl.ds(0, _LANES)] = v0
    v1 = filt[pl.ds(_LANES, _LANES)]
    v1 = 1.0 / (1.0 + jnp.exp(-v1))
    filt[pl.ds(_LANES, _LANES)] = v1

    def chunk_body(ci, n):
      _sync_copy(pts_hbm.at[pl.ds(ci * 4 * chunk, 4 * chunk)], pbuf)

      def vec_body(i, n):
        off = i * _LANES
        xv = pbuf[pl.ds(off, _LANES)]
        yv = pbuf[pl.ds(chunk + off, _LANES)]
        zv = pbuf[pl.ds(2 * chunk + off, _LANES)]
        cv = pbuf[pl.ds(3 * chunk + off, _LANES)]
        fx = (xv - minb[0]) * inv_vs[0]
        fy = (yv - minb[1]) * inv_vs[1]
        fz = (zv - minb[2]) * inv_vs[2]
        ix = jnp.clip(fx.astype(jnp.int32), 0, X - 1)
        iy = jnp.clip(fy.astype(jnp.int32), 0, Y - 1)
        iz = jnp.clip(fz.astype(jnp.int32), 0, Z - 1)
        ic = jnp.clip(cv.astype(jnp.int32), 0, C - 1)
        valid = ((xv >= minb[0]) & (xv < maxb[0])
                 & (yv >= minb[1]) & (yv < maxb[1])
                 & (zv >= minb[2]) & (zv < maxb[2]))
        m = valid & (ix >= win_lo) & (ix <= x_lo + xpw)
        code = ((ix - win_lo) << 18) | (iy << 10) | (iz << 5) | ic
        mi = m.astype(jnp.int32)
        pos = jnp.clip(n + _sc_cumsum(mi) - 1, 0, clcap - 1)
        _sc_store_scatter(clist, [pos], code, m)
        return n + jnp.sum(mi)

      return lax.fori_loop(0, vecs_per_chunk, vec_body, n)

    n_pts = lax.fori_loop(0, n_chunks, chunk_body, jnp.int32(0))

    def slab_body(s, _):
      def filt_body(i, ns):
        off = i * _LANES
        codes = clist[pl.ds(off, _LANES)]
        lm = (lane_iota() + off) < n_pts
        ixl = codes >> 18
        m = lm & (ixl >= s) & (ixl <= s + 2)
        mi = m.astype(jnp.int32)
        pos = jnp.clip(ns + _sc_cumsum(mi) - 1, 0, slcap - 1)
        _sc_store_scatter(slist, [pos], codes, m)
        return ns + jnp.sum(mi)

      n_vecs = (n_pts + _LANES - 1) // _LANES
      ns_pts = lax.fori_loop(0, n_vecs, filt_body, jnp.int32(0))
      sx = x_lo + s

      def half_body(h, _):
        rbase = (sx * 2 + h) * reg
        _sync_copy(map_hbm.at[pl.ds(rbase, reg)], acc)
        ylo = h * yh

        def pt_body(i, _):
          off = i * _LANES
          codes = slist[pl.ds(off, _LANES)]
          lm = (lane_iota() + off) < ns_pts
          ixl = codes >> 18
          iy = (codes >> 10) & 0xFF
          iz = (codes >> 5) & 0x1F
          ic = codes & 0x1F
          ly = iy - ylo
          m0 = lm & (ly >= -1) & (ly <= yh)
          bidx = ly * zc + iz * C + ic
          cnts, _lastm = _sc_scan_count(bidx, m0)
          minc = jnp.min(jnp.where(m0, cnts, jnp.int32(2**30)))
          maxc = jnp.max(jnp.where(m0, cnts, jnp.int32(-2**30)))
          k9 = ((ixl - 1 - s) + 1) * 9
          my = {dy: (ly + dy >= 0) & (ly + dy < yh) for dy in (-1, 0, 1)}
          mz = {dz: (iz + dz >= 0) & (iz + dz < Z) for dz in (-1, 0, 1)}
          taps = []
          for dy in (-1, 0, 1):
            for dz in (-1, 0, 1):
              mt = m0 & my[dy] & mz[dz]
              wv = _sc_load_gather(
                  filt, jnp.clip(k9 + (1 - dy) * 3 + (1 - dz), 0, 31))
              tidx = jnp.clip(bidx + (dy * zc + dz * C), 0, reg - 1)
              taps.append((tidx, wv, mt))

          def round_body(r, _):
            mr = cnts == r
            for tidx, wv, mt in taps:
              _sc_scatter_add(acc, [tidx], wv, mt & mr)
            return 0

          lax.fori_loop(minc, maxc + 1, round_body, 0)
          return 0

        ns_vecs = (ns_pts + _LANES - 1) // _LANES
        lax.fori_loop(0, ns_vecs, pt_body, 0)
        _sync_copy(acc, out_hbm.at[pl.ds(rbase, reg)])
        return 0

      lax.fori_loop(0, 2, half_body, 0)
      return 0

    lax.fori_loop(0, xpw, slab_body, 0)

  return body


def _make_kernel(X, Y, Z, C, n_pad, chunk, clcap, slcap):
  body = _make_body(X, Y, Z, C, n_pad, chunk, clcap, slcap)
  mesh = plsc.VectorSubcoreMesh(
      core_axis_name="c", subcore_axis_name="s", num_cores=_NC,
      num_subcores=_NS)
  return pl.kernel(
      body,
      out_type=jax.ShapeDtypeStruct((X * Y * Z * C,), jnp.float32),
      mesh=mesh,
      scratch_types=[
          pltpu.VMEM(((Y // 2) * Z * C,), jnp.float32),
          pltpu.VMEM((clcap,), jnp.int32),
          pltpu.VMEM((slcap,), jnp.int32),
          pltpu.VMEM((4 * chunk,), jnp.float32),
          pltpu.VMEM((32,), jnp.float32),
      ],
      compiler_params=pltpu.CompilerParams(
          needs_layout_passes=False, use_tc_tiling_on_sc=False),
  )


@jax.jit
def kernel(current_map, point_cloud, weights):
  X, Y, Z, C = current_map.shape
  n = point_cloud.shape[0]
  chunk = 2048
  n_pad = ((n + chunk - 1) // chunk) * chunk
  pts = jnp.concatenate(
      [point_cloud,
       jnp.full((n_pad - n, 4), 1e30, point_cloud.dtype)], axis=0)
  ptsf = jnp.transpose(
      pts.reshape(n_pad // chunk, chunk, 4), (0, 2, 1)).reshape(-1)
  w_flat = jnp.concatenate(
      [weights.reshape(-1), jnp.zeros((32 - 27,), weights.dtype)])
  k = _make_kernel(X, Y, Z, C, n_pad, chunk, clcap=16000, slcap=8192)
  return k(current_map.reshape(-1), ptsf, w_flat).reshape(X, Y, Z, C)


# speedup vs baseline: 7.5230x; 7.0397x over previous
import functools

import jax
import jax.numpy as jnp
import numpy as np
from jax import lax
from jax.experimental import pallas as pl
from jax.experimental.pallas import tpu as pltpu
from jax.experimental.pallas import tpu_sc as plsc

_GRID = (256, 256, 32)
_NUM_CLASSES = 21
_MIN_B = np.array([-25.6, -25.6, -2.0], np.float32)
_MAX_B = np.array([25.6, 25.6, 4.4], np.float32)

_NC = 2
_NS = 16
_NW = _NC * _NS
_LANES = 16

_SCB = 1


def _sc_scatter_add(ref, idxs, x, mask):
  plsc.addupdate_scatter(ref, idxs, x, mask=mask)


def _sc_store_scatter(ref, idxs, x, mask):
  plsc.store_scatter(ref, idxs, x, mask=mask)


def _sc_load_gather(ref, idx):
  return plsc.load_gather(ref, [idx])


def _sc_scan_count(x, mask):
  return plsc.scan_count(x, mask=mask)


def _axis_index(name):
  return lax.axis_index(name)


def _sync_copy(src, dst):
  pltpu.sync_copy(src, dst)


def _make_body(X, Y, Z, C, n_pad, chunk, ccap):
  assert X % _NW == 0 and Y % 128 == 0 and Z % 8 == 0 and n_pad % chunk == 0
  assert chunk % _LANES == 0
  xpw = X // _NW
  yb = Y // 128
  plane = (Z // 8) * yb * 1024
  n_chunks = n_pad // chunk
  vecs_per_chunk = chunk // _LANES

  minb = [float(v) for v in _MIN_B]
  maxb = [float(v) for v in _MAX_B]
  vs = (np.asarray(_MAX_B) - np.asarray(_MIN_B)) / np.asarray(
      (X, Y, Z), np.float32)
  inv_vs = [float(np.float32(1.0) / v) for v in vs]

  lane_iota = lambda: lax.iota(jnp.int32, _LANES)

  def tiled_idx(z, y):
    return (((z >> 3) * yb + (y >> 7)) << 10) | ((z & 7) << 7) | (y & 127)

  def body(map_hbm, pts_hbm, w_hbm, out_hbm, acc, bkts, cnt, pbuf, filt):
    wid = _axis_index("s") * _NC + _axis_index("c")
    x_lo = wid * xpw
    win_lo = x_lo - 1

    _sync_copy(w_hbm, filt)
    v0 = filt[pl.ds(0, _LANES)]
    v0 = 1.0 / (1.0 + jnp.exp(-v0))
    v0 = jnp.where(lane_iota() == 13, 1.0, v0)
    filt[pl.ds(0, _LANES)] = v0
    v1 = filt[pl.ds(_LANES, _LANES)]
    v1 = 1.0 / (1.0 + jnp.exp(-v1))
    filt[pl.ds(_LANES, _LANES)] = v1

    zeros = jnp.zeros((_LANES,), jnp.int32)
    cnt[pl.ds(0, _LANES)] = zeros
    cnt[pl.ds(_LANES, _LANES)] = zeros

    def chunk_body(ci, carry):
      _sync_copy(pts_hbm.at[pl.ds(ci * 4 * chunk, 4 * chunk)], pbuf)

      def vec_body(i, carry):
        off = i * _LANES
        xv = pbuf[pl.ds(off, _LANES)]
        yv = pbuf[pl.ds(chunk + off, _LANES)]
        zv = pbuf[pl.ds(2 * chunk + off, _LANES)]
        cv = pbuf[pl.ds(3 * chunk + off, _LANES)]
        fx = (xv - minb[0]) * inv_vs[0]
        fy = (yv - minb[1]) * inv_vs[1]
        fz = (zv - minb[2]) * inv_vs[2]
        ix = jnp.clip(fx.astype(jnp.int32), 0, X - 1)
        iy = jnp.clip(fy.astype(jnp.int32), 0, Y - 1)
        iz = jnp.clip(fz.astype(jnp.int32), 0, Z - 1)
        ic = jnp.clip(cv.astype(jnp.int32), 0, C - 1)
        valid = ((xv >= minb[0]) & (xv < maxb[0])
                 & (yv >= minb[1]) & (yv < maxb[1])
                 & (zv >= minb[2]) & (zv < maxb[2]))
        m = valid & (ix >= win_lo) & (ix <= x_lo + xpw)
        code = ((ix - win_lo) << 13) | (iy << 5) | iz
        cnts, lastm = _sc_scan_count(ic, m)
        base = _sc_load_gather(cnt, ic)
        pos = jnp.clip(base + (cnts - _SCB), 0, ccap - 1)
        _sc_store_scatter(bkts, [ic * ccap + pos], code, m)
        _sc_scatter_add(cnt, [ic], cnts - _SCB + 1, m & lastm)
        return carry

      return lax.fori_loop(0, vecs_per_chunk, vec_body, carry)

    lax.fori_loop(0, n_chunks, chunk_body, 0)

    def slab_body(s, _):
      sx = x_lo + s

      def class_body(c, _):
        rbase = (sx * C + c) * plane
        _sync_copy(map_hbm.at[pl.ds(rbase, plane)], acc)
        nc = jnp.max(_sc_load_gather(cnt, jnp.full((_LANES,), c, jnp.int32)))

        def pt_body(i, _):
          off = i * _LANES
          codes = bkts[pl.ds(c * ccap + off, _LANES)]
          lm = (lane_iota() + off) < nc
          ixl = codes >> 13
          iy = (codes >> 5) & 0xFF
          iz = codes & 0x1F
          m0 = lm & (ixl >= s) & (ixl <= s + 2)
          bidx = tiled_idx(iz, iy)
          cnts, _lastm = _sc_scan_count(bidx, m0)
          minc = jnp.min(jnp.where(m0, cnts, jnp.int32(2**30)))
          maxc = jnp.max(jnp.where(m0, cnts, jnp.int32(-2**30)))
          k9 = (ixl - s) * 9
          taps = []
          for dy in (-1, 0, 1):
            ty = iy + dy
            my = (ty >= 0) & (ty < Y)
            for dz in (-1, 0, 1):
              tz = iz + dz
              mt = m0 & my & (tz >= 0) & (tz < Z)
              wv = _sc_load_gather(
                  filt, jnp.clip(k9 + (1 - dy) * 3 + (1 - dz), 0, 31))
              tidx = jnp.clip(tiled_idx(tz, ty), 0, plane - 1)
              taps.append((tidx, wv, mt))

          def round_body(r, _):
            mr = cnts == r
            for tidx, wv, mt in taps:
              _sc_scatter_add(acc, [tidx], wv, mt & mr)
            return 0

          lax.fori_loop(minc, maxc + 1, round_body, 0)
          return 0

        nvec = (nc + _LANES - 1) // _LANES
        lax.fori_loop(0, nvec, pt_body, 0)
        _sync_copy(acc, out_hbm.at[pl.ds(rbase, plane)])
        return 0

      lax.fori_loop(0, C, class_body, 0)
      return 0

    lax.fori_loop(0, xpw, slab_body, 0)

  return body


def _make_kernel(X, Y, Z, C, n_pad, chunk, ccap):
  body = _make_body(X, Y, Z, C, n_pad, chunk, ccap)
  plane = (Z // 8) * (Y // 128) * 1024
  mesh = plsc.VectorSubcoreMesh(
      core_axis_name="c", subcore_axis_name="s", num_cores=_NC,
      num_subcores=_NS)
  return pl.kernel(
      body,
      out_type=jax.ShapeDtypeStruct((X * Y * Z * C,), jnp.float32),
      mesh=mesh,
      scratch_types=[
          pltpu.VMEM((plane,), jnp.float32),
          pltpu.VMEM((C * ccap,), jnp.int32),
          pltpu.VMEM((32,), jnp.int32),
          pltpu.VMEM((4 * chunk,), jnp.float32),
          pltpu.VMEM((32,), jnp.float32),
      ],
      compiler_params=pltpu.CompilerParams(
          needs_layout_passes=False, use_tc_tiling_on_sc=False),
  )


@jax.jit
def kernel(current_map, point_cloud, weights):
  X, Y, Z, C = current_map.shape
  n = point_cloud.shape[0]
  chunk = 2048
  n_pad = ((n + chunk - 1) // chunk) * chunk
  pts = jnp.concatenate(
      [point_cloud,
       jnp.full((n_pad - n, 4), 1e30, point_cloud.dtype)], axis=0)
  ptsf = jnp.transpose(
      pts.reshape(n_pad // chunk, chunk, 4), (0, 2, 1)).reshape(-1)
  w_flat = jnp.concatenate(
      [weights.reshape(-1), jnp.zeros((32 - 27,), weights.dtype)])
  flat_b = (current_map.transpose(0, 3, 2, 1)
            .reshape(X, C, Z // 8, 8, Y // 128, 128)
            .transpose(0, 1, 2, 4, 3, 5).reshape(-1))
  k = _make_kernel(X, Y, Z, C, n_pad, chunk, ccap=2048)
  of = k(flat_b, ptsf, w_flat)
  out = (of.reshape(X, C, Z // 8, Y // 128, 8, 128)
         .transpose(0, 1, 2, 4, 3, 5)
         .reshape(X, C, Z, Y).transpose(0, 3, 2, 1))
  return out


# speedup vs baseline: 10.7051x; 1.4230x over previous
import functools

import jax
import jax.numpy as jnp
import numpy as np
from jax import lax
from jax.experimental import pallas as pl
from jax.experimental.pallas import tpu as pltpu
from jax.experimental.pallas import tpu_sc as plsc

_GRID = (256, 256, 32)
_NUM_CLASSES = 21
_MIN_B = np.array([-25.6, -25.6, -2.0], np.float32)
_MAX_B = np.array([25.6, 25.6, 4.4], np.float32)

_NC = 2
_NS = 16
_NW = _NC * _NS
_LANES = 16

_SCB = 1


def _sc_scatter_add(ref, idxs, x, mask):
  plsc.addupdate_scatter(ref, idxs, x, mask=mask)


def _sc_store_scatter(ref, idxs, x, mask):
  plsc.store_scatter(ref, idxs, x, mask=mask)


def _sc_load_gather(ref, idx):
  return plsc.load_gather(ref, [idx])


def _sc_scan_count(x, mask):
  return plsc.scan_count(x, mask=mask)


def _axis_index(name):
  return lax.axis_index(name)


def _sync_copy(src, dst):
  pltpu.sync_copy(src, dst)


def _copy_start(src, dst, sem):
  pltpu.make_async_copy(src, dst, sem).start()


def _copy_wait(src, dst, sem):
  pltpu.make_async_copy(src, dst, sem).wait()


def _make_body(X, Y, Z, C, n_pad, chunk, ccap):
  assert X % _NW == 0 and Y % 128 == 0 and Z % 8 == 0 and n_pad % chunk == 0
  assert chunk % _LANES == 0
  xpw = X // _NW
  yb = Y // 128
  plane = (Z // 8) * yb * 1024
  n_chunks = n_pad // chunk
  vecs_per_chunk = chunk // _LANES

  minb = [float(v) for v in _MIN_B]
  maxb = [float(v) for v in _MAX_B]
  vs = (np.asarray(_MAX_B) - np.asarray(_MIN_B)) / np.asarray(
      (X, Y, Z), np.float32)
  inv_vs = [float(np.float32(1.0) / v) for v in vs]

  lane_iota = lambda: lax.iota(jnp.int32, _LANES)

  def tiled_idx(z, y):
    return (((z >> 3) * yb + (y >> 7)) << 10) | ((z & 7) << 7) | (y & 127)

  def body(map_hbm, pts_hbm, w_hbm, out_hbm, acc0, acc1, bkts, cnt,
           pbuf0, pbuf1, filt, sin0, sin1, sout0, sout1, spt0, spt1):
    accs = (acc0, acc1)
    sins = (sin0, sin1)
    souts = (sout0, sout1)
    pbufs = (pbuf0, pbuf1)
    spts = (spt0, spt1)
    wid = _axis_index("s") * _NC + _axis_index("c")
    x_lo = wid * xpw
    win_lo = x_lo - 1

    _sync_copy(w_hbm, filt)
    v0 = filt[pl.ds(0, _LANES)]
    v0 = 1.0 / (1.0 + jnp.exp(-v0))
    v0 = jnp.where(lane_iota() == 13, 1.0, v0)
    filt[pl.ds(0, _LANES)] = v0
    v1 = filt[pl.ds(_LANES, _LANES)]
    v1 = 1.0 / (1.0 + jnp.exp(-v1))
    filt[pl.ds(_LANES, _LANES)] = v1

    zeros = jnp.zeros((_LANES,), jnp.int32)
    cnt[pl.ds(0, _LANES)] = zeros
    cnt[pl.ds(_LANES, _LANES)] = zeros

    def start_pin(ci, b):
      _copy_start(pts_hbm.at[pl.ds(ci * 4 * chunk, 4 * chunk)],
                  pbufs[b], spts[b])

    def wait_pin(ci, b):
      _copy_wait(pts_hbm.at[pl.ds(ci * 4 * chunk, 4 * chunk)],
                 pbufs[b], spts[b])

    def chunk_work(ci, b, carry):
      pbuf = pbufs[b]

      def vec_body(i, carry):
        off = i * _LANES
        xv = pbuf[pl.ds(off, _LANES)]
        yv = pbuf[pl.ds(chunk + off, _LANES)]
        zv = pbuf[pl.ds(2 * chunk + off, _LANES)]
        cv = pbuf[pl.ds(3 * chunk + off, _LANES)]
        fx = (xv - minb[0]) * inv_vs[0]
        fy = (yv - minb[1]) * inv_vs[1]
        fz = (zv - minb[2]) * inv_vs[2]
        ix = jnp.clip(fx.astype(jnp.int32), 0, X - 1)
        iy = jnp.clip(fy.astype(jnp.int32), 0, Y - 1)
        iz = jnp.clip(fz.astype(jnp.int32), 0, Z - 1)
        ic = jnp.clip(cv.astype(jnp.int32), 0, C - 1)
        valid = ((xv >= minb[0]) & (xv < maxb[0])
                 & (yv >= minb[1]) & (yv < maxb[1])
                 & (zv >= minb[2]) & (zv < maxb[2]))
        m = valid & (ix >= win_lo) & (ix <= x_lo + xpw)
        code = ((ix - win_lo) << 13) | (iy << 5) | iz
        cnts, lastm = _sc_scan_count(ic, m)
        base = _sc_load_gather(cnt, ic)
        pos = jnp.clip(base + (cnts - _SCB), 0, ccap - 1)
        _sc_store_scatter(bkts, [ic * ccap + pos], code, m)
        _sc_scatter_add(cnt, [ic], cnts - _SCB + 1, m & lastm)
        return carry

      return lax.fori_loop(0, vecs_per_chunk, vec_body, carry)

    def pchunk_pair(p, carry):
      c0 = 2 * p

      @pl.when(c0 + 1 < n_chunks)
      def _():
        start_pin(c0 + 1, 1)

      wait_pin(c0, 0)
      carry = chunk_work(c0, 0, carry)
      c1 = c0 + 1

      @pl.when(c1 + 1 < n_chunks)
      def _():
        start_pin(c1 + 1, 0)

      def do_c1(carry):
        wait_pin(c1, 1)
        return chunk_work(c1, 1, carry)

      return lax.cond(c1 < n_chunks, do_c1, lambda car: car, carry)

    start_pin(0, 0)
    lax.fori_loop(0, (n_chunks + 1) // 2, pchunk_pair, 0)

    nreg = xpw * C

    def rslice(s, c):
      return pl.ds(((x_lo + s) * C + c) * plane, plane)

    def start_in(s, c, b):
      _copy_start(map_hbm.at[rslice(s, c)], accs[b], sins[b])

    def wait_in(s, c, b):
      _copy_wait(map_hbm.at[rslice(s, c)], accs[b], sins[b])

    def start_out(s, c, b):
      _copy_start(accs[b], out_hbm.at[rslice(s, c)], souts[b])

    def wait_out(s, c, b):
      _copy_wait(accs[b], out_hbm.at[rslice(s, c)], souts[b])

    def inc(s, c):
      wrap = c + 1 == C
      return s + wrap, jnp.where(wrap, 0, c + 1)

    def process(s, c, b):
      acc = accs[b]
      nc = jnp.max(_sc_load_gather(cnt, jnp.full((_LANES,), c, jnp.int32)))

      def pt_body(i, _):
        off = i * _LANES
        codes = bkts[pl.ds(c * ccap + off, _LANES)]
        lm = (lane_iota() + off) < nc
        ixl = codes >> 13
        iy = (codes >> 5) & 0xFF
        iz = codes & 0x1F
        m0 = lm & (ixl >= s) & (ixl <= s + 2)
        bidx = tiled_idx(iz, iy)
        cnts, _lastm = _sc_scan_count(bidx, m0)
        minc = jnp.min(jnp.where(m0, cnts, jnp.int32(2**30)))
        maxc = jnp.max(jnp.where(m0, cnts, jnp.int32(-2**30)))
        k9 = (ixl - s) * 9
        taps = []
        for dy in (-1, 0, 1):
          ty = iy + dy
          my = (ty >= 0) & (ty < Y)
          for dz in (-1, 0, 1):
            tz = iz + dz
            mt = m0 & my & (tz >= 0) & (tz < Z)
            wv = _sc_load_gather(
                filt, jnp.clip(k9 + (1 - dy) * 3 + (1 - dz), 0, 31))
            tidx = jnp.clip(tiled_idx(tz, ty), 0, plane - 1)
            taps.append((tidx, wv, mt))

        def round_body(r, _):
          mr = cnts == r
          for tidx, wv, mt in taps:
            _sc_scatter_add(acc, [tidx], wv, mt & mr)
          return 0

        lax.fori_loop(minc, maxc + 1, round_body, 0)
        return 0

      nvec = (nc + _LANES - 1) // _LANES
      lax.fori_loop(0, nvec, pt_body, 0)

    def pair_body(p, sc):
      s0, c0 = sc
      r0 = 2 * p
      s1, c1 = inc(s0, c0)
      s2, c2 = inc(s1, c1)

      @pl.when(r0 + 1 < nreg)
      def _():
        @pl.when(r0 >= 1)
        def _():
          wait_out(s0, c0, 1)

        start_in(s1, c1, 1)

      wait_in(s0, c0, 0)
      process(s0, c0, 0)
      start_out(s0, c0, 0)

      @pl.when(r0 + 1 < nreg)
      def _():
        wait_in(s1, c1, 1)
        process(s1, c1, 1)
        start_out(s1, c1, 1)

        @pl.when(r0 + 2 < nreg)
        def _():
          wait_out(s0, c0, 0)
          start_in(s2, c2, 0)

      return s2, c2

    start_in(0, 0, 0)
    lax.fori_loop(0, (nreg + 1) // 2, pair_body,
                  (jnp.int32(0), jnp.int32(0)))
    wait_out(0, 0, (nreg - 1) % 2)
    if nreg >= 2:
      wait_out(0, 0, (nreg - 2) % 2)

  return body


def _make_kernel(X, Y, Z, C, n_pad, chunk, ccap):
  body = _make_body(X, Y, Z, C, n_pad, chunk, ccap)
  plane = (Z // 8) * (Y // 128) * 1024
  mesh = plsc.VectorSubcoreMesh(
      core_axis_name="c", subcore_axis_name="s", num_cores=_NC,
      num_subcores=_NS)
  return pl.kernel(
      body,
      out_type=jax.ShapeDtypeStruct((X * Y * Z * C,), jnp.float32),
      mesh=mesh,
      scratch_types=[
          pltpu.VMEM((plane,), jnp.float32),
          pltpu.VMEM((plane,), jnp.float32),
          pltpu.VMEM((C * ccap,), jnp.int32),
          pltpu.VMEM((32,), jnp.int32),
          pltpu.VMEM((4 * chunk,), jnp.float32),
          pltpu.VMEM((4 * chunk,), jnp.float32),
          pltpu.VMEM((32,), jnp.float32),
          pltpu.SemaphoreType.DMA,
          pltpu.SemaphoreType.DMA,
          pltpu.SemaphoreType.DMA,
          pltpu.SemaphoreType.DMA,
          pltpu.SemaphoreType.DMA,
          pltpu.SemaphoreType.DMA,
      ],
      compiler_params=pltpu.CompilerParams(
          needs_layout_passes=False, use_tc_tiling_on_sc=False),
  )


@jax.jit
def kernel(current_map, point_cloud, weights):
  X, Y, Z, C = current_map.shape
  n = point_cloud.shape[0]
  chunk = 2048
  n_pad = ((n + chunk - 1) // chunk) * chunk
  pts = jnp.concatenate(
      [point_cloud,
       jnp.full((n_pad - n, 4), 1e30, point_cloud.dtype)], axis=0)
  ptsf = jnp.transpose(
      pts.reshape(n_pad // chunk, chunk, 4), (0, 2, 1)).reshape(-1)
  w_flat = jnp.concatenate(
      [weights.reshape(-1), jnp.zeros((32 - 27,), weights.dtype)])
  flat_b = (current_map.transpose(0, 3, 2, 1)
            .reshape(X, C, Z // 8, 8, Y // 128, 128)
            .transpose(0, 1, 2, 4, 3, 5).reshape(-1))
  k = _make_kernel(X, Y, Z, C, n_pad, chunk, ccap=2048)
  of = k(flat_b, ptsf, w_flat)
  out = (of.reshape(X, C, Z // 8, Y // 128, 8, 128)
         .transpose(0, 1, 2, 4, 3, 5)
         .reshape(X, C, Z, Y).transpose(0, 3, 2, 1))
  return out


# speedup vs baseline: 11.3671x; 1.0618x over previous
import functools

import jax
import jax.numpy as jnp
import numpy as np
from jax import lax
from jax.experimental import pallas as pl
from jax.experimental.pallas import tpu as pltpu
from jax.experimental.pallas import tpu_sc as plsc

_GRID = (256, 256, 32)
_NUM_CLASSES = 21
_MIN_B = np.array([-25.6, -25.6, -2.0], np.float32)
_MAX_B = np.array([25.6, 25.6, 4.4], np.float32)

_NC = 2
_NS = 16
_NW = _NC * _NS
_LANES = 16

_SCB = 1


def _sc_scatter_add(ref, idxs, x, mask):
  plsc.addupdate_scatter(ref, idxs, x, mask=mask)


def _sc_store_scatter(ref, idxs, x, mask):
  plsc.store_scatter(ref, idxs, x, mask=mask)


def _sc_load_gather(ref, idx):
  return plsc.load_gather(ref, [idx])


def _sc_scan_count(x, mask):
  return plsc.scan_count(x, mask=mask)


def _sc_cumsum(x):
  return plsc.cumsum(x)


def _axis_index(name):
  return lax.axis_index(name)


def _sync_copy(src, dst):
  pltpu.sync_copy(src, dst)


def _copy_start(src, dst, sem):
  pltpu.make_async_copy(src, dst, sem).start()


def _copy_wait(src, dst, sem):
  pltpu.make_async_copy(src, dst, sem).wait()


def _make_body(X, Y, Z, C, n_pad, chunk, ccap, clcap):
  assert X % _NW == 0 and Y % 128 == 0 and Z % 8 == 0 and n_pad % chunk == 0
  assert chunk % _LANES == 0
  xpw = X // _NW
  yb = Y // 128
  plane = (Z // 8) * yb * 1024
  n_chunks = n_pad // chunk
  vecs_per_chunk = chunk // _LANES

  minb = [float(v) for v in _MIN_B]
  maxb = [float(v) for v in _MAX_B]
  vs = (np.asarray(_MAX_B) - np.asarray(_MIN_B)) / np.asarray(
      (X, Y, Z), np.float32)
  inv_vs = [float(np.float32(1.0) / v) for v in vs]

  lane_iota = lambda: lax.iota(jnp.int32, _LANES)

  def tiled_idx(z, y):
    return (((z >> 3) * yb + (y >> 7)) << 10) | ((z & 7) << 7) | (y & 127)

  def body(map_hbm, pts_hbm, w_hbm, out_hbm, acc0, acc1, bkts, cnt, clist,
           pbuf0, pbuf1, filt, sin0, sin1, sout0, sout1, spt0, spt1):
    accs = (acc0, acc1)
    sins = (sin0, sin1)
    souts = (sout0, sout1)
    pbufs = (pbuf0, pbuf1)
    spts = (spt0, spt1)
    wid = _axis_index("s") * _NC + _axis_index("c")
    x_lo = wid * xpw
    win_lo = x_lo - 1

    _sync_copy(w_hbm, filt)
    v0 = filt[pl.ds(0, _LANES)]
    v0 = 1.0 / (1.0 + jnp.exp(-v0))
    v0 = jnp.where(lane_iota() == 13, 1.0, v0)
    filt[pl.ds(0, _LANES)] = v0
    v1 = filt[pl.ds(_LANES, _LANES)]
    v1 = 1.0 / (1.0 + jnp.exp(-v1))
    filt[pl.ds(_LANES, _LANES)] = v1

    zeros = jnp.zeros((_LANES,), jnp.int32)
    cnt[pl.ds(0, _LANES)] = zeros
    cnt[pl.ds(_LANES, _LANES)] = zeros

    def start_pin(ci, b):
      _copy_start(pts_hbm.at[pl.ds(ci * 4 * chunk, 4 * chunk)],
                  pbufs[b], spts[b])

    def wait_pin(ci, b):
      _copy_wait(pts_hbm.at[pl.ds(ci * 4 * chunk, 4 * chunk)],
                 pbufs[b], spts[b])

    def chunk_work(ci, b, carry):
      pbuf = pbufs[b]

      def vec_body(i, carry):
        off = i * _LANES
        xv = pbuf[pl.ds(off, _LANES)]
        yv = pbuf[pl.ds(chunk + off, _LANES)]
        zv = pbuf[pl.ds(2 * chunk + off, _LANES)]
        cv = pbuf[pl.ds(3 * chunk + off, _LANES)]
        fx = (xv - minb[0]) * inv_vs[0]
        fy = (yv - minb[1]) * inv_vs[1]
        fz = (zv - minb[2]) * inv_vs[2]
        ix = jnp.clip(fx.astype(jnp.int32), 0, X - 1)
        iy = jnp.clip(fy.astype(jnp.int32), 0, Y - 1)
        iz = jnp.clip(fz.astype(jnp.int32), 0, Z - 1)
        ic = jnp.clip(cv.astype(jnp.int32), 0, C - 1)
        valid = ((xv >= minb[0]) & (xv < maxb[0])
                 & (yv >= minb[1]) & (yv < maxb[1])
                 & (zv >= minb[2]) & (zv < maxb[2]))
        m = valid & (ix >= win_lo) & (ix <= x_lo + xpw)
        code = ((ix - win_lo) << 18) | (iy << 10) | (iz << 5) | ic
        mi = m.astype(jnp.int32)
        pos = jnp.clip(carry + _sc_cumsum(mi) - 1, 0, clcap - 1)
        _sc_store_scatter(clist, [pos], code, m)
        return carry + jnp.sum(mi)

      return lax.fori_loop(0, vecs_per_chunk, vec_body, carry)

    def pchunk_pair(p, carry):
      c0 = 2 * p

      @pl.when(c0 + 1 < n_chunks)
      def _():
        start_pin(c0 + 1, 1)

      wait_pin(c0, 0)
      carry = chunk_work(c0, 0, carry)
      c1 = c0 + 1

      @pl.when(c1 + 1 < n_chunks)
      def _():
        start_pin(c1 + 1, 0)

      def do_c1(carry):
        wait_pin(c1, 1)
        return chunk_work(c1, 1, carry)

      return lax.cond(c1 < n_chunks, do_c1, lambda car: car, carry)

    start_pin(0, 0)
    n_pts = lax.fori_loop(0, (n_chunks + 1) // 2, pchunk_pair, jnp.int32(0))

    def bucket_body(i, _):
      off = i * _LANES
      codes = clist[pl.ds(off, _LANES)]
      m = (lane_iota() + off) < n_pts
      ic = jnp.clip(codes & 0x1F, 0, C - 1)
      cnts, lastm = _sc_scan_count(ic, m)
      base = _sc_load_gather(cnt, ic)
      pos = jnp.clip(base + (cnts - _SCB), 0, ccap - 1)
      _sc_store_scatter(bkts, [ic * ccap + pos], codes >> 5, m)
      _sc_scatter_add(cnt, [ic], cnts - _SCB + 1, m & lastm)
      return 0

    lax.fori_loop(0, (n_pts + _LANES - 1) // _LANES, bucket_body, 0)

    nreg = xpw * C

    def rslice(s, c):
      return pl.ds(((x_lo + s) * C + c) * plane, plane)

    def start_in(s, c, b):
      _copy_start(map_hbm.at[rslice(s, c)], accs[b], sins[b])

    def wait_in(s, c, b):
      _copy_wait(map_hbm.at[rslice(s, c)], accs[b], sins[b])

    def start_out(s, c, b):
      _copy_start(accs[b], out_hbm.at[rslice(s, c)], souts[b])

    def wait_out(s, c, b):
      _copy_wait(accs[b], out_hbm.at[rslice(s, c)], souts[b])

    def inc(s, c):
      wrap = c + 1 == C
      return s + wrap, jnp.where(wrap, 0, c + 1)

    def process(s, c, b):
      acc = accs[b]
      nc = jnp.max(_sc_load_gather(cnt, jnp.full((_LANES,), c, jnp.int32)))

      def pt_body(i, _):
        off = i * _LANES
        codes = bkts[pl.ds(c * ccap + off, _LANES)]
        lm = (lane_iota() + off) < nc
        ixl = codes >> 13
        iy = (codes >> 5) & 0xFF
        iz = codes & 0x1F
        m0 = lm & (ixl >= s) & (ixl <= s + 2)
        bidx = tiled_idx(iz, iy)
        cnts, _lastm = _sc_scan_count(bidx, m0)
        minc = jnp.min(jnp.where(m0, cnts, jnp.int32(2**30)))
        maxc = jnp.max(jnp.where(m0, cnts, jnp.int32(-2**30)))
        k9 = (ixl - s) * 9
        taps = []
        for dy in (-1, 0, 1):
          ty = iy + dy
          my = (ty >= 0) & (ty < Y)
          for dz in (-1, 0, 1):
            tz = iz + dz
            mt = m0 & my & (tz >= 0) & (tz < Z)
            wv = _sc_load_gather(
                filt, jnp.clip(k9 + (1 - dy) * 3 + (1 - dz), 0, 31))
            tidx = jnp.clip(tiled_idx(tz, ty), 0, plane - 1)
            taps.append((tidx, wv, mt))

        def round_body(r, _):
          mr = cnts == r
          for tidx, wv, mt in taps:
            _sc_scatter_add(acc, [tidx], wv, mt & mr)
          return 0

        lax.fori_loop(minc, maxc + 1, round_body, 0)
        return 0

      nvec = (nc + _LANES - 1) // _LANES
      lax.fori_loop(0, nvec, pt_body, 0)

    def pair_body(p, sc):
      s0, c0 = sc
      r0 = 2 * p
      s1, c1 = inc(s0, c0)
      s2, c2 = inc(s1, c1)

      @pl.when(r0 + 1 < nreg)
      def _():
        @pl.when(r0 >= 1)
        def _():
          wait_out(s0, c0, 1)

        start_in(s1, c1, 1)

      wait_in(s0, c0, 0)
      process(s0, c0, 0)
      start_out(s0, c0, 0)

      @pl.when(r0 + 1 < nreg)
      def _():
        wait_in(s1, c1, 1)
        process(s1, c1, 1)
        start_out(s1, c1, 1)

        @pl.when(r0 + 2 < nreg)
        def _():
          wait_out(s0, c0, 0)
          start_in(s2, c2, 0)

      return s2, c2

    start_in(0, 0, 0)
    lax.fori_loop(0, (nreg + 1) // 2, pair_body,
                  (jnp.int32(0), jnp.int32(0)))
    wait_out(0, 0, (nreg - 1) % 2)
    if nreg >= 2:
      wait_out(0, 0, (nreg - 2) % 2)

  return body


def _make_kernel(X, Y, Z, C, n_pad, chunk, ccap, clcap):
  body = _make_body(X, Y, Z, C, n_pad, chunk, ccap, clcap)
  plane = (Z // 8) * (Y // 128) * 1024
  mesh = plsc.VectorSubcoreMesh(
      core_axis_name="c", subcore_axis_name="s", num_cores=_NC,
      num_subcores=_NS)
  return pl.kernel(
      body,
      out_type=jax.ShapeDtypeStruct((X * Y * Z * C,), jnp.float32),
      mesh=mesh,
      scratch_types=[
          pltpu.VMEM((plane,), jnp.float32),
          pltpu.VMEM((plane,), jnp.float32),
          pltpu.VMEM((C * ccap,), jnp.int32),
          pltpu.VMEM((32,), jnp.int32),
          pltpu.VMEM((clcap,), jnp.int32),
          pltpu.VMEM((4 * chunk,), jnp.float32),
          pltpu.VMEM((4 * chunk,), jnp.float32),
          pltpu.VMEM((32,), jnp.float32),
          pltpu.SemaphoreType.DMA,
          pltpu.SemaphoreType.DMA,
          pltpu.SemaphoreType.DMA,
          pltpu.SemaphoreType.DMA,
          pltpu.SemaphoreType.DMA,
          pltpu.SemaphoreType.DMA,
      ],
      compiler_params=pltpu.CompilerParams(
          needs_layout_passes=False, use_tc_tiling_on_sc=False),
  )


@jax.jit
def kernel(current_map, point_cloud, weights):
  X, Y, Z, C = current_map.shape
  n = point_cloud.shape[0]
  chunk = 2048
  n_pad = ((n + chunk - 1) // chunk) * chunk
  pts = jnp.concatenate(
      [point_cloud,
       jnp.full((n_pad - n, 4), 1e30, point_cloud.dtype)], axis=0)
  ptsf = jnp.transpose(
      pts.reshape(n_pad // chunk, chunk, 4), (0, 2, 1)).reshape(-1)
  w_flat = jnp.concatenate(
      [weights.reshape(-1), jnp.zeros((32 - 27,), weights.dtype)])
  flat_b = (current_map.transpose(0, 3, 2, 1)
            .reshape(X, C, Z // 8, 8, Y // 128, 128)
            .transpose(0, 1, 2, 4, 3, 5).reshape(-1))
  k = _make_kernel(X, Y, Z, C, n_pad, chunk, ccap=2048, clcap=16000)
  of = k(flat_b, ptsf, w_flat)
  out = (of.reshape(X, C, Z // 8, Y // 128, 8, 128)
         .transpose(0, 1, 2, 4, 3, 5)
         .reshape(X, C, Z, Y).transpose(0, 3, 2, 1))
  return out
